# baseline (speedup 1.0000x reference)
import functools

import jax
import jax.numpy as jnp
from jax import lax
from jax.experimental import pallas as pl
from jax.experimental.pallas import tpu as pltpu
from jax.experimental.pallas import tpu_sc as plsc

D_MODEL = 256
T_ROWS = 256
NUM_CORES = 2
NUM_SUBCORES = 16
NUM_WORKERS = NUM_CORES * NUM_SUBCORES
ROW = 128
NBUF = 2


def _prep_kernel(card_ref, stg_ref, rank_ref, suit_ref, stage_ref,
                 t_ref, idx_ref):
    rows = lax.broadcasted_iota(jnp.int32, (T_ROWS, 1), 0)
    card = rows // 4
    stg = rows % 4
    rank = card % 13
    suit = card // 13
    valid = card < 52
    acc = jnp.zeros((T_ROWS, D_MODEL), jnp.float32)
    for k in range(13):
        acc += jnp.where(rank == k, 1.0, 0.0) * rank_ref[k, :][None, :]
    for k in range(4):
        acc += jnp.where(suit == k, 1.0, 0.0) * suit_ref[k, :][None, :]
        acc += jnp.where(stg == k, 1.0, 0.0) * stage_ref[k, :][None, :]
    t_ref[...] = jnp.where(valid, acc, 0.0)

    c = card_ref[...]
    s = stg_ref[...]
    cvalid = (c >= 0) & (c < 52)
    cc = jnp.where(cvalid, c, 52)
    ss = jnp.clip(s, 0, 3)
    idx_ref[...] = cc * 4 + ss


def _prep(card2, stg2, rank_emb, suit_emb, stage_emb):
    n_rows, row = card2.shape
    return pl.pallas_call(
        _prep_kernel,
        out_shape=(
            jax.ShapeDtypeStruct((T_ROWS, D_MODEL), jnp.float32),
            jax.ShapeDtypeStruct((n_rows, row), jnp.int32),
        ),
    )(card2, stg2, rank_emb, suit_emb, stage_emb)


def _make_sc_gather(n_rows):
    assert n_rows % (NUM_WORKERS * NBUF) == 0
    rows_per_worker = n_rows // NUM_WORKERS
    mesh = plsc.VectorSubcoreMesh(core_axis_name="c", subcore_axis_name="s")

    scratch = []
    for _ in range(NBUF):
        scratch += [
            pltpu.VMEM((ROW,), jnp.int32),
            pltpu.VMEM((ROW, D_MODEL), jnp.float32),
            pltpu.SemaphoreType.DMA,
            pltpu.SemaphoreType.DMA,
            pltpu.SemaphoreType.DMA,
        ]

    @functools.partial(
        pl.kernel,
        out_type=jax.ShapeDtypeStruct((n_rows * ROW, D_MODEL), jnp.float32),
        mesh=mesh,
        scratch_types=scratch,
    )
    def sc_gather(idx_hbm, t_hbm, out_hbm, *bufs):
        idx_v = [bufs[5 * b + 0] for b in range(NBUF)]
        rows_v = [bufs[5 * b + 1] for b in range(NBUF)]
        isem = [bufs[5 * b + 2] for b in range(NBUF)]
        gsem = [bufs[5 * b + 3] for b in range(NBUF)]
        osem = [bufs[5 * b + 4] for b in range(NBUF)]
        wid = lax.axis_index("s") * NUM_CORES + lax.axis_index("c")
        row0 = wid * rows_per_worker

        for b in range(NBUF):
            pltpu.async_copy(idx_hbm.at[row0 + b], idx_v[b], isem[b])

        def group(g, carry):
            for b in range(NBUF):
                r = g * NBUF + b
                pltpu.make_async_copy(
                    idx_hbm.at[row0], idx_v[b], isem[b]).wait()

                @pl.when(r >= NBUF)
                def _rows_free():
                    pltpu.make_async_copy(
                        rows_v[b], out_hbm.at[pl.ds(0, ROW)], osem[b]).wait()

                pltpu.async_copy(t_hbm.at[idx_v[b]], rows_v[b], gsem[b]).wait()

                @pl.when(r + NBUF < rows_per_worker)
                def _next_idx():
                    pltpu.async_copy(
                        idx_hbm.at[row0 + r + NBUF], idx_v[b], isem[b])

                pltpu.async_copy(
                    rows_v[b], out_hbm.at[pl.ds((row0 + r) * ROW, ROW)],
                    osem[b])
            return carry

        lax.fori_loop(0, rows_per_worker // NBUF, group, 0)
        for b in range(NBUF):
            pltpu.make_async_copy(
                rows_v[b], out_hbm.at[pl.ds(0, ROW)], osem[b]).wait()

    return sc_gather


def kernel(card_indices, stages, rank_emb, suit_emb, stage_emb):
    batch, seq = card_indices.shape
    n_pos = batch * seq
    n_rows = n_pos // ROW
    card2 = card_indices.astype(jnp.int32).reshape(n_rows, ROW)
    stg2 = stages.astype(jnp.int32).reshape(n_rows, ROW)
    table, idx = _prep(card2, stg2, rank_emb, suit_emb, stage_emb)
    out = _make_sc_gather(n_rows)(idx, table)
    return out.reshape(batch, seq, D_MODEL)
